# baseline (speedup 1.0000x reference)
import functools

import jax
import jax.numpy as jnp
from jax import lax
from jax.experimental import pallas as pl
from jax.experimental.pallas import tpu as pltpu
from jax.experimental.pallas import tpu_sc as plsc

NC = 2
NS = 16
NW = NC * NS
GL = 128
K = 8
CH = K * GL

TBLK = 4096


def _eye(n):
    i = lax.broadcasted_iota(jnp.int32, (n, n), 0)
    j = lax.broadcasted_iota(jnp.int32, (n, n), 1)
    return (i == j).astype(jnp.float32)


def _tpose_wide_body(x_ref, o_ref):
    x = x_ref[...]
    o_ref[...] = lax.dot_general(
        x, _eye(x.shape[0]), (((0,), (0,)), ((), ())),
        preferred_element_type=jnp.float32,
    )


def _tpose_tall_body(x_ref, o_ref):
    x = x_ref[...]
    o_ref[...] = lax.dot_general(
        _eye(x.shape[1]), x, (((1,), (1,)), ((), ())),
        preferred_element_type=jnp.float32,
    )


@functools.lru_cache(maxsize=None)
def _tpose_wide(rows, cols, blk):
    nblk = pl.cdiv(cols, blk)
    return pl.pallas_call(
        _tpose_wide_body,
        grid=(nblk,),
        in_specs=[pl.BlockSpec((rows, blk), lambda i: (0, i))],
        out_specs=pl.BlockSpec((blk, rows), lambda i: (i, 0)),
        out_shape=jax.ShapeDtypeStruct((cols, rows), jnp.float32),
    )


@functools.lru_cache(maxsize=None)
def _tpose_tall(rows, cols, blk):
    nblk = pl.cdiv(rows, blk)
    return pl.pallas_call(
        _tpose_tall_body,
        grid=(nblk,),
        in_specs=[pl.BlockSpec((blk, cols), lambda i: (i, 0))],
        out_specs=pl.BlockSpec((cols, blk), lambda i: (0, i)),
        out_shape=jax.ShapeDtypeStruct((cols, rows), jnp.float32),
    )


@functools.lru_cache(maxsize=None)
def _gather(v, d, nch):
    mesh = plsc.VectorSubcoreMesh(core_axis_name="c", subcore_axis_name="s")

    @functools.partial(
        pl.kernel,
        mesh=mesh,
        out_type=jax.ShapeDtypeStruct((NW, nch, CH, d), jnp.float32),
        scratch_types=[
            pltpu.VMEM((nch * K, GL), jnp.int32),
            pltpu.VMEM((CH, d), jnp.float32),
            pltpu.SemaphoreType.DMA,
        ],
        compiler_params=pltpu.CompilerParams(use_tc_tiling_on_sc=False),
    )
    def k(table_hbm, tok_hbm, out_hbm, idx_v, rows_v, sem):
        wid = lax.axis_index("s") * NC + lax.axis_index("c")
        pltpu.sync_copy(tok_hbm.at[wid], idx_v)

        def chunk(c, carry):
            cps = [
                pltpu.async_copy(
                    table_hbm.at[idx_v.at[c * K + j]],
                    rows_v.at[pl.ds(j * GL, GL)],
                    sem,
                )
                for j in range(K)
            ]
            for cp in cps:
                cp.wait()
            pltpu.sync_copy(rows_v, out_hbm.at[wid, c])
            return carry

        lax.fori_loop(0, nch, chunk, 0)

    return k


def kernel(tokens, table):
    v, d = table.shape
    flat = tokens.reshape(-1).astype(jnp.int32)
    b = flat.shape[0]
    blk = NW * CH
    pad = (-b) % blk
    if pad:
        flat = jnp.concatenate([flat, jnp.zeros((pad,), jnp.int32)])
    nch = flat.shape[0] // blk
    tok3 = flat.reshape(NW, nch * K, GL)

    table_rm = _tpose_wide(d, v, TBLK)(table.T)
    rows = _gather(v, d, nch)(table_rm, tok3)
    rows2 = rows.reshape(-1, d)
    out_t = _tpose_tall(rows2.shape[0], d, TBLK)(rows2)
    out = out_t.T
    if pad:
        out = out[:b]
    return out


# speedup vs baseline: 1.1986x; 1.1986x over previous
import functools

import jax
import jax.numpy as jnp
from jax import lax
from jax.experimental import pallas as pl
from jax.experimental.pallas import tpu as pltpu
from jax.experimental.pallas import tpu_sc as plsc

NC = 2
NS = 16
NW = NC * NS
GL = 128
K = 8
CH = K * GL

BLKA = 1024
BLKB = 1024


def _pack_body(x_ref, o_ref):
    x = x_ref[...]
    blk = x.shape[1] // 2
    o_ref[:, 0:64] = x[:, 0:blk].T
    o_ref[:, 64:128] = x[:, blk:].T


def _unpack_body(x_ref, o_ref):
    x = x_ref[...]
    blk = x.shape[0]
    o_ref[:, 0:blk] = x[:, 0:64].T
    o_ref[:, blk:] = x[:, 64:128].T


@functools.lru_cache(maxsize=None)
def _pack(cols, blk):
    nblk = pl.cdiv(cols // 2, blk)
    return pl.pallas_call(
        _pack_body,
        grid=(nblk,),
        in_specs=[pl.BlockSpec((64, 2 * blk), lambda i: (0, i))],
        out_specs=pl.BlockSpec((blk, 128), lambda i: (i, 0)),
        out_shape=jax.ShapeDtypeStruct((nblk * blk, 128), jnp.float32),
    )


@functools.lru_cache(maxsize=None)
def _unpack(rows, blk):
    nblk = pl.cdiv(rows, blk)
    return pl.pallas_call(
        _unpack_body,
        grid=(nblk,),
        in_specs=[pl.BlockSpec((blk, 128), lambda i: (i, 0))],
        out_specs=pl.BlockSpec((64, 2 * blk), lambda i: (0, i)),
        out_shape=jax.ShapeDtypeStruct((64, 2 * rows), jnp.float32),
    )


@functools.lru_cache(maxsize=None)
def _gather(v, d, nch):
    mesh = plsc.VectorSubcoreMesh(core_axis_name="c", subcore_axis_name="s")

    @functools.partial(
        pl.kernel,
        mesh=mesh,
        out_type=jax.ShapeDtypeStruct((NW, nch, CH, d), jnp.float32),
        scratch_types=[
            pltpu.VMEM((nch * K, GL), jnp.int32),
            pltpu.VMEM((CH, d), jnp.float32),
            pltpu.SemaphoreType.DMA,
        ],
        compiler_params=pltpu.CompilerParams(use_tc_tiling_on_sc=False),
    )
    def k(table_hbm, tok_hbm, out_hbm, idx_v, rows_v, sem):
        wid = lax.axis_index("s") * NC + lax.axis_index("c")
        pltpu.sync_copy(tok_hbm.at[wid], idx_v)

        def chunk(c, carry):
            cps = [
                pltpu.async_copy(
                    table_hbm.at[idx_v.at[c * K + j]],
                    rows_v.at[pl.ds(j * GL, GL)],
                    sem,
                )
                for j in range(K)
            ]
            for cp in cps:
                cp.wait()
            pltpu.sync_copy(rows_v, out_hbm.at[wid, c])
            return carry

        lax.fori_loop(0, nch, chunk, 0)

    return k


def _kernel_fast(tokens, table):
    v, d = table.shape
    flat = tokens.reshape(-1).astype(jnp.int32)
    b = flat.shape[0]

    packed = _pack(v, BLKA)(table.T)
    v_lin = 2 * packed.shape[0]
    table_lin = packed.reshape(-1).reshape(v_lin, d)

    u = flat % (2 * BLKA)
    slots = (flat - u) + 2 * (u % BLKA) + u // BLKA

    nb = b // (2 * BLKB)
    p = slots.reshape(nb, 2, BLKB).transpose(0, 2, 1).reshape(-1)

    blk = NW * CH
    nch = b // blk
    tok3 = p.reshape(NW, nch * K, GL)

    rows = _gather(v_lin, d, nch)(table_lin, tok3)

    out_t = _unpack(b // 2, BLKB)(rows.reshape(-1).reshape(b // 2, 128))
    return out_t.T


def _kernel_simple(tokens, table):
    v, d = table.shape
    flat = tokens.reshape(-1).astype(jnp.int32)
    b = flat.shape[0]
    blk = NW * CH
    pad = (-b) % blk
    if pad:
        flat = jnp.concatenate([flat, jnp.zeros((pad,), jnp.int32)])
    nch = flat.shape[0] // blk
    tok3 = flat.reshape(NW, nch * K, GL)
    out = _gather(v, d, nch)(table, tok3)
    out = out.reshape(-1, d)
    if pad:
        out = out[:b]
    return out


def kernel(tokens, table):
    v, d = table.shape
    b = tokens.size
    if d == 64 and v % 2 == 0 and b % (NW * CH) == 0 and b % (2 * BLKB) == 0:
        return _kernel_fast(tokens, table)
    return _kernel_simple(tokens, table)


# speedup vs baseline: 1.4806x; 1.2352x over previous
import functools

import jax
import jax.numpy as jnp
from jax import lax
from jax.experimental import pallas as pl
from jax.experimental.pallas import tpu as pltpu
from jax.experimental.pallas import tpu_sc as plsc

NC = 2
NS = 16
NW = NC * NS
GL = 128
K = 8
CH = K * GL

BLKA = 1024
BLKB = 1024


def _pack_body(x_ref, o_ref):
    x = x_ref[...]
    blk = x.shape[1] // 2
    o_ref[:, 0:64] = x[:, 0:blk].T
    o_ref[:, 64:128] = x[:, blk:].T


def _unpack_body(x_ref, o_ref):
    x = x_ref[...]
    blk = x.shape[0]
    for w in range(blk // 64):
        t = x[64 * w:64 * (w + 1), :].T
        o_ref[:, 128 * w:128 * w + 64] = t[0:64, :]
        o_ref[:, 128 * w + 64:128 * (w + 1)] = t[64:128, :]


@functools.lru_cache(maxsize=None)
def _pack(cols, blk):
    nblk = pl.cdiv(cols // 2, blk)
    return pl.pallas_call(
        _pack_body,
        grid=(nblk,),
        in_specs=[pl.BlockSpec((64, 2 * blk), lambda i: (0, i))],
        out_specs=pl.BlockSpec((blk, 128), lambda i: (i, 0)),
        out_shape=jax.ShapeDtypeStruct((nblk * blk, 128), jnp.float32),
    )


@functools.lru_cache(maxsize=None)
def _unpack(rows, blk):
    nblk = pl.cdiv(rows, blk)
    return pl.pallas_call(
        _unpack_body,
        grid=(nblk,),
        in_specs=[pl.BlockSpec((blk, 128), lambda i: (i, 0))],
        out_specs=pl.BlockSpec((64, 2 * blk), lambda i: (0, i)),
        out_shape=jax.ShapeDtypeStruct((64, 2 * rows), jnp.float32),
    )


@functools.lru_cache(maxsize=None)
def _gather(v, d, nch, perm=True):
    mesh = plsc.VectorSubcoreMesh(core_axis_name="c", subcore_axis_name="s")

    @functools.partial(
        pl.kernel,
        mesh=mesh,
        out_type=jax.ShapeDtypeStruct((NW, nch, CH, d), jnp.float32),
        scratch_types=[
            pltpu.VMEM((nch * K * GL,), jnp.int32),
            pltpu.VMEM((nch * K * GL,), jnp.int32),
            pltpu.VMEM((CH, d), jnp.float32),
            pltpu.SemaphoreType.DMA,
        ],
        compiler_params=pltpu.CompilerParams(
            use_tc_tiling_on_sc=False,
            needs_layout_passes=False,
        ),
    )
    def k(table_hbm, tok_hbm, out_hbm, idx_v, idxp_v, rows_v, sem):
        wid = lax.axis_index("s") * NC + lax.axis_index("c")
        pltpu.sync_copy(tok_hbm.at[wid], idx_v)

        i16 = lax.iota(jnp.int32, 16)
        patt = 64 * (i16 % 2) + i16 // 2

        def permrow(r, carry):
            base = r * GL
            for t in range(8):
                if perm:
                    x = plsc.load_gather(idx_v, [base + patt + 8 * t])
                else:
                    x = idx_v[pl.ds(base + 16 * t, 16)]
                idxp_v[pl.ds(base + 16 * t, 16)] = x
            return carry

        lax.fori_loop(0, nch * K, permrow, 0)

        def chunk(c, carry):
            cps = [
                pltpu.async_copy(
                    table_hbm.at[idxp_v.at[pl.ds((c * K + j) * GL, GL)]],
                    rows_v.at[pl.ds(j * GL, GL)],
                    sem,
                )
                for j in range(K)
            ]
            for cp in cps:
                cp.wait()
            pltpu.sync_copy(rows_v, out_hbm.at[wid, c])
            return carry

        lax.fori_loop(0, nch, chunk, 0)

    return k


def _kernel_fast(tokens, table):
    v, d = table.shape
    flat = tokens.reshape(-1).astype(jnp.int32)
    b = flat.shape[0]

    packed = _pack(v, BLKA)(table.T)
    v_lin = 2 * packed.shape[0]
    table_lin = packed.reshape(-1).reshape(v_lin, d)

    u = flat % (2 * BLKA)
    slots = (flat - u) + 2 * (u % BLKA) + u // BLKA

    blk = NW * CH
    nch = b // blk
    tok3 = slots.reshape(NW, nch * K * GL)

    rows = _gather(v_lin, d, nch)(table_lin, tok3)

    out_t = _unpack(b // 2, BLKB)(rows.reshape(-1).reshape(b // 2, 128))
    return out_t.T


def _kernel_simple(tokens, table):
    v, d = table.shape
    flat = tokens.reshape(-1).astype(jnp.int32)
    b = flat.shape[0]
    blk = NW * CH
    pad = (-b) % blk
    if pad:
        flat = jnp.concatenate([flat, jnp.zeros((pad,), jnp.int32)])
    nch = flat.shape[0] // blk
    tok3 = flat.reshape(NW, nch * K * GL)
    out = _gather(v, d, nch, perm=False)(table, tok3)
    out = out.reshape(-1, d)
    if pad:
        out = out[:b]
    return out


def kernel(tokens, table):
    v, d = table.shape
    b = tokens.size
    if d == 64 and v % 2 == 0 and b % (NW * CH) == 0 and b % (2 * BLKB) == 0:
        return _kernel_fast(tokens, table)
    return _kernel_simple(tokens, table)
